# baseline (speedup 1.0000x reference)
import functools

import jax
import jax.numpy as jnp
from jax import lax
from jax.experimental import pallas as pl
from jax.experimental.pallas import tpu as pltpu
from jax.experimental.pallas import tpu_sc as plsc

N = 10000
E = 320000
D = 128
G = 128

NW = 32
CH = 128
CPW = 80
EP = NW * CPW * CH
NH = N + 8
RPT = 624
TAIL = N - 16 * RPT
HB = CPW // 2


def _sc_scatter(h, src2, dst2, zrows):
    mesh = plsc.VectorSubcoreMesh(core_axis_name="c", subcore_axis_name="s")

    @functools.partial(
        pl.kernel,
        mesh=mesh,
        out_type=(
            jax.ShapeDtypeStruct((N, D), jnp.float32),
            jax.ShapeDtypeStruct((N, D), jnp.float32),
        ),
        scratch_types=[
            pltpu.VMEM((HB, CH), jnp.int32),
            pltpu.VMEM((HB, CH), jnp.int32),
            pltpu.VMEM((CH, D), jnp.float32),
            pltpu.VMEM((CH, D), jnp.float32),
            pltpu.VMEM_SHARED((N, D), jnp.float32),
            pltpu.SemaphoreType.DMA,
            pltpu.SemaphoreType.DMA,
        ],
    )
    def k(h_hbm, src_hbm, dst_hbm, z_hbm, out0, out1,
          si, di, buf0, buf1, agg_sh, gs0, gs1):
        c = lax.axis_index("c")
        s = lax.axis_index("s")
        wid = s * 2 + c

        stripe = s * RPT
        pltpu.sync_copy(z_hbm, agg_sh.at[pl.ds(stripe, RPT)])

        @pl.when(s == 15)
        def _():
            pltpu.sync_copy(z_hbm.at[pl.ds(0, TAIL)],
                            agg_sh.at[pl.ds(16 * RPT, TAIL)])

        plsc.subcore_barrier()

        for half in range(CPW // HB):
            pltpu.sync_copy(src_hbm.at[pl.ds(wid * CPW + half * HB, HB)], si)
            pltpu.sync_copy(dst_hbm.at[pl.ds(wid * CPW + half * HB, HB)], di)
            def chunk(j, carry):
                pltpu.async_copy(h_hbm.at[si.at[j]], buf0, gs0).wait()
                pltpu.sync_copy(buf0, agg_sh.at[di.at[j]], add=True)
                return carry

            lax.fori_loop(0, HB, chunk, 0)
        plsc.subcore_barrier()

        @pl.when(c == 0)
        def _():
            pltpu.sync_copy(agg_sh.at[pl.ds(stripe, RPT)],
                            out0.at[pl.ds(stripe, RPT)])

            @pl.when(s == 15)
            def _():
                pltpu.sync_copy(agg_sh.at[pl.ds(16 * RPT, TAIL)],
                                out0.at[pl.ds(16 * RPT, TAIL)])

        @pl.when(c == 1)
        def _():
            pltpu.sync_copy(agg_sh.at[pl.ds(stripe, RPT)],
                            out1.at[pl.ds(stripe, RPT)])

            @pl.when(s == 15)
            def _():
                pltpu.sync_copy(agg_sh.at[pl.ds(16 * RPT, TAIL)],
                                out1.at[pl.ds(16 * RPT, TAIL)])

    return k(h, src2, dst2, zrows)


R = 1000
GRID = N // R


def _mlp_stats_body(x_r, a0_r, a1_r, w1_r, b1_r, w2_r, b2_r,
                    t_r, sum_r, sq_r, accs, accq):
    i = pl.program_id(0)
    m = x_r[...] + a0_r[...] + a1_r[...]
    hmid = jnp.maximum(
        jnp.dot(m, w1_r[...], preferred_element_type=jnp.float32) + b1_r[...],
        0.0)
    t = jnp.dot(hmid, w2_r[...], preferred_element_type=jnp.float32) + b2_r[...]
    t_r[...] = t
    t3 = t.reshape(R // 8, 8, D)
    ps = jnp.sum(t3, axis=0)
    pq = jnp.sum(t3 * t3, axis=0)

    @pl.when(i == 0)
    def _():
        accs[...] = ps
        accq[...] = pq

    @pl.when(i > 0)
    def _():
        accs[...] += ps
        accq[...] += pq

    @pl.when(i == GRID - 1)
    def _():
        sum_r[...] = accs[...]
        sq_r[...] = accq[...]


def _mlp_stats(x, a0, a1, w1, b1, w2, b2):
    blk = pl.BlockSpec((R, D), lambda i: (i, 0))
    full = pl.BlockSpec((D, D), lambda i: (0, 0))
    vec = pl.BlockSpec((D,), lambda i: (0,))
    return pl.pallas_call(
        _mlp_stats_body,
        grid=(GRID,),
        in_specs=[blk, blk, blk, full, vec, full, vec],
        out_specs=[blk,
                   pl.BlockSpec((8, D), lambda i: (0, 0)),
                   pl.BlockSpec((8, D), lambda i: (0, 0))],
        out_shape=[jax.ShapeDtypeStruct((N, D), jnp.float32),
                   jax.ShapeDtypeStruct((8, D), jnp.float32),
                   jax.ShapeDtypeStruct((8, D), jnp.float32)],
        scratch_shapes=[pltpu.VMEM((8, D), jnp.float32),
                        pltpu.VMEM((8, D), jnp.float32)],
    )(x, a0, a1, w1, b1, w2, b2)


def _bn_relu_body(t_r, sum_r, sq_r, g_r, be_r, h_r):
    mean = jnp.sum(sum_r[...], axis=0) / N
    ex2 = jnp.sum(sq_r[...], axis=0) / N
    var = ex2 - mean * mean
    scale = lax.rsqrt(var + 1e-5) * g_r[...]
    h_r[...] = jnp.maximum((t_r[...] - mean) * scale + be_r[...], 0.0)


def _bn_relu(t, s8, q8, g, be):
    blk = pl.BlockSpec((R, D), lambda i: (i, 0))
    stat = pl.BlockSpec((8, D), lambda i: (0, 0))
    vec = pl.BlockSpec((D,), lambda i: (0,))
    return pl.pallas_call(
        _bn_relu_body,
        grid=(GRID,),
        in_specs=[blk, stat, stat, vec, vec],
        out_specs=blk,
        out_shape=jax.ShapeDtypeStruct((N, D), jnp.float32),
    )(t, s8, q8, g, be)


def _pool_body(h_r, b_r, wc_r, bc_r, out_r, accp, accc):
    i = pl.program_id(0)
    bvec = b_r[0, 0, :]
    gids = lax.broadcasted_iota(jnp.int32, (G, R), 0)
    mask = (gids == bvec[None, :]).astype(jnp.float32)
    ps = jnp.dot(mask, h_r[...], preferred_element_type=jnp.float32)
    pc = jnp.dot(mask, jnp.ones((R, D), jnp.float32),
                 preferred_element_type=jnp.float32)

    @pl.when(i == 0)
    def _():
        accp[...] = ps
        accc[...] = pc

    @pl.when(i > 0)
    def _():
        accp[...] += ps
        accc[...] += pc

    @pl.when(i == GRID - 1)
    def _():
        pooled = accp[...] / jnp.maximum(accc[...], 1.0)
        out_r[...] = (jnp.dot(pooled, wc_r[...],
                              preferred_element_type=jnp.float32) + bc_r[...])


def _pool(h, batch3, wc, bc):
    blk = pl.BlockSpec((R, D), lambda i: (i, 0))
    full = pl.BlockSpec((D, D), lambda i: (0, 0))
    vec = pl.BlockSpec((D,), lambda i: (0,))
    return pl.pallas_call(
        _pool_body,
        grid=(GRID,),
        in_specs=[blk,
                  pl.BlockSpec((1, 1, R), lambda i: (i, 0, 0)),
                  full, vec],
        out_specs=pl.BlockSpec((G, D), lambda i: (0, 0)),
        out_shape=jax.ShapeDtypeStruct((G, D), jnp.float32),
        scratch_shapes=[pltpu.VMEM((G, D), jnp.float32),
                        pltpu.VMEM((G, D), jnp.float32)],
    )(h, batch3, wc, bc)


def kernel(x, edge_index, batch,
           W1_0, b1_0, W2_0, b2_0, g_0, be_0,
           W1_1, b1_1, W2_1, b2_1, g_1, be_1,
           W1_2, b1_2, W2_2, b2_2, g_2, be_2,
           Wc, bc):
    params = [
        (W1_0, b1_0, W2_0, b2_0, g_0, be_0),
        (W1_1, b1_1, W2_1, b2_1, g_1, be_1),
        (W1_2, b1_2, W2_2, b2_2, g_2, be_2),
    ]
    pad = jnp.zeros((EP - E,), jnp.int32)
    src2 = jnp.concatenate([edge_index[0], pad + N]).reshape(EP // CH, CH)
    dst2 = jnp.concatenate([edge_index[1], pad]).reshape(EP // CH, CH)
    hrow_pad = jnp.zeros((NH - N, D), jnp.float32)
    zrows = jnp.zeros((RPT, D), jnp.float32)
    h = x
    for (w1, b1, w2, b2, g, be) in params:
        h_ext = jnp.concatenate([h, hrow_pad], axis=0)
        a0, a1 = _sc_scatter(h_ext, src2, dst2, zrows)
        t, s8, q8 = _mlp_stats(h, a0, a1, w1, b1, w2, b2)
        h = _bn_relu(t, s8, q8, g, be)
    batch3 = batch.reshape(GRID, 1, R)
    return _pool(h, batch3, Wc, bc)


# speedup vs baseline: 1.1120x; 1.1120x over previous
import functools

import jax
import jax.numpy as jnp
from jax import lax
from jax.experimental import pallas as pl
from jax.experimental.pallas import tpu as pltpu
from jax.experimental.pallas import tpu_sc as plsc

N = 10000
E = 320000
D = 128
G = 128

NW = 32
CH = 128
CPW = 80
EP = NW * CPW * CH
NH = N + 8
RPT = 624
TAIL = N - 16 * RPT
HB = CPW // 2


def _sc_scatter(h, src1, dst1, zrows):
    mesh = plsc.VectorSubcoreMesh(core_axis_name="c", subcore_axis_name="s")

    @functools.partial(
        pl.kernel,
        mesh=mesh,
        out_type=(
            jax.ShapeDtypeStruct((N, D), jnp.float32),
            jax.ShapeDtypeStruct((N, D), jnp.float32),
        ),
        scratch_types=(
            [pltpu.VMEM((CH,), jnp.int32)] * 8
            + [pltpu.VMEM((CH, D), jnp.float32)] * 2
            + [pltpu.VMEM_SHARED((N, D), jnp.float32)]
            + [pltpu.SemaphoreType.DMA] * 6
        ),
    )
    def k(h_hbm, src_hbm, dst_hbm, z_hbm, out0, out1,
          si0, si1, si2, si3, di0, di1, di2, di3, buf0, buf1, agg_sh,
          is0, is1, is2, is3, gs0, gs1):
        c = lax.axis_index("c")
        s = lax.axis_index("s")
        wid = s * 2 + c
        sis = [si0, si1, si2, si3]
        dis = [di0, di1, di2, di3]
        bufs = [buf0, buf1]
        iss = [is0, is1, is2, is3]
        gss = [gs0, gs1]

        stripe = s * RPT
        pltpu.sync_copy(z_hbm, agg_sh.at[pl.ds(stripe, RPT)])

        @pl.when(s == 15)
        def _():
            pltpu.sync_copy(z_hbm.at[pl.ds(0, TAIL)],
                            agg_sh.at[pl.ds(16 * RPT, TAIL)])

        plsc.subcore_barrier()

        ebase = wid * (CPW * CH)

        def idx_start(m, t):
            pltpu.async_copy(src_hbm.at[pl.ds(ebase + m * CH, CH)],
                             sis[t], iss[t])
            pltpu.async_copy(dst_hbm.at[pl.ds(ebase + m * CH, CH)],
                             dis[t], iss[t])

        def idx_wait(m, t):
            pltpu.make_async_copy(src_hbm.at[pl.ds(ebase + m * CH, CH)],
                                  sis[t], iss[t]).wait()
            pltpu.make_async_copy(dst_hbm.at[pl.ds(ebase + m * CH, CH)],
                                  dis[t], iss[t]).wait()

        def gat_start(t, b):
            pltpu.async_copy(h_hbm.at[sis[t]], bufs[b], gss[b])

        def gat_wait(t, b):
            pltpu.make_async_copy(h_hbm.at[sis[t]], bufs[b], gss[b]).wait()

        for t in range(4):
            idx_start(t, t)
        idx_wait(0, 0)
        gat_start(0, 0)

        def body(j, carry):
            for kk in range(4):
                m = 4 * j + kk
                t = kk
                b = kk % 2
                tn = (kk + 1) % 4
                bn = (kk + 1) % 2

                @pl.when(m < CPW - 1)
                def _():
                    idx_wait(m + 1, tn)
                    gat_start(tn, bn)

                gat_wait(t, b)
                pltpu.sync_copy(bufs[b], agg_sh.at[dis[t]], add=True)

                @pl.when(m < CPW - 4)
                def _():
                    idx_start(m + 4, t)
            return carry

        lax.fori_loop(0, CPW // 4, body, 0)
        plsc.subcore_barrier()

        @pl.when(c == 0)
        def _():
            pltpu.sync_copy(agg_sh.at[pl.ds(stripe, RPT)],
                            out0.at[pl.ds(stripe, RPT)])

            @pl.when(s == 15)
            def _():
                pltpu.sync_copy(agg_sh.at[pl.ds(16 * RPT, TAIL)],
                                out0.at[pl.ds(16 * RPT, TAIL)])

        @pl.when(c == 1)
        def _():
            pltpu.sync_copy(agg_sh.at[pl.ds(stripe, RPT)],
                            out1.at[pl.ds(stripe, RPT)])

            @pl.when(s == 15)
            def _():
                pltpu.sync_copy(agg_sh.at[pl.ds(16 * RPT, TAIL)],
                                out1.at[pl.ds(16 * RPT, TAIL)])

    return k(h, src1, dst1, zrows)


R = 1000
GRID = N // R


def _mlp_stats_body(x_r, a0_r, a1_r, w1_r, b1_r, w2_r, b2_r,
                    t_r, sum_r, sq_r, accs, accq):
    i = pl.program_id(0)
    m = x_r[...] + a0_r[...] + a1_r[...]
    hmid = jnp.maximum(
        jnp.dot(m, w1_r[...], preferred_element_type=jnp.float32) + b1_r[...],
        0.0)
    t = jnp.dot(hmid, w2_r[...], preferred_element_type=jnp.float32) + b2_r[...]
    t_r[...] = t
    t3 = t.reshape(R // 8, 8, D)
    ps = jnp.sum(t3, axis=0)
    pq = jnp.sum(t3 * t3, axis=0)

    @pl.when(i == 0)
    def _():
        accs[...] = ps
        accq[...] = pq

    @pl.when(i > 0)
    def _():
        accs[...] += ps
        accq[...] += pq

    @pl.when(i == GRID - 1)
    def _():
        sum_r[...] = accs[...]
        sq_r[...] = accq[...]


def _mlp_stats(x, a0, a1, w1, b1, w2, b2):
    blk = pl.BlockSpec((R, D), lambda i: (i, 0))
    full = pl.BlockSpec((D, D), lambda i: (0, 0))
    vec = pl.BlockSpec((D,), lambda i: (0,))
    return pl.pallas_call(
        _mlp_stats_body,
        grid=(GRID,),
        in_specs=[blk, blk, blk, full, vec, full, vec],
        out_specs=[blk,
                   pl.BlockSpec((8, D), lambda i: (0, 0)),
                   pl.BlockSpec((8, D), lambda i: (0, 0))],
        out_shape=[jax.ShapeDtypeStruct((N, D), jnp.float32),
                   jax.ShapeDtypeStruct((8, D), jnp.float32),
                   jax.ShapeDtypeStruct((8, D), jnp.float32)],
        scratch_shapes=[pltpu.VMEM((8, D), jnp.float32),
                        pltpu.VMEM((8, D), jnp.float32)],
    )(x, a0, a1, w1, b1, w2, b2)


def _bn_relu_body(t_r, sum_r, sq_r, g_r, be_r, h_r):
    mean = jnp.sum(sum_r[...], axis=0) / N
    ex2 = jnp.sum(sq_r[...], axis=0) / N
    var = ex2 - mean * mean
    scale = lax.rsqrt(var + 1e-5) * g_r[...]
    h_r[...] = jnp.maximum((t_r[...] - mean) * scale + be_r[...], 0.0)


def _bn_relu(t, s8, q8, g, be):
    blk = pl.BlockSpec((R, D), lambda i: (i, 0))
    stat = pl.BlockSpec((8, D), lambda i: (0, 0))
    vec = pl.BlockSpec((D,), lambda i: (0,))
    return pl.pallas_call(
        _bn_relu_body,
        grid=(GRID,),
        in_specs=[blk, stat, stat, vec, vec],
        out_specs=blk,
        out_shape=jax.ShapeDtypeStruct((N, D), jnp.float32),
    )(t, s8, q8, g, be)


def _pool_body(h_r, b_r, wc_r, bc_r, out_r, accp, accc):
    i = pl.program_id(0)
    bvec = b_r[0, 0, :]
    gids = lax.broadcasted_iota(jnp.int32, (G, R), 0)
    mask = (gids == bvec[None, :]).astype(jnp.float32)
    ps = jnp.dot(mask, h_r[...], preferred_element_type=jnp.float32)
    pc = jnp.dot(mask, jnp.ones((R, D), jnp.float32),
                 preferred_element_type=jnp.float32)

    @pl.when(i == 0)
    def _():
        accp[...] = ps
        accc[...] = pc

    @pl.when(i > 0)
    def _():
        accp[...] += ps
        accc[...] += pc

    @pl.when(i == GRID - 1)
    def _():
        pooled = accp[...] / jnp.maximum(accc[...], 1.0)
        out_r[...] = (jnp.dot(pooled, wc_r[...],
                              preferred_element_type=jnp.float32) + bc_r[...])


def _pool(h, batch3, wc, bc):
    blk = pl.BlockSpec((R, D), lambda i: (i, 0))
    full = pl.BlockSpec((D, D), lambda i: (0, 0))
    vec = pl.BlockSpec((D,), lambda i: (0,))
    return pl.pallas_call(
        _pool_body,
        grid=(GRID,),
        in_specs=[blk,
                  pl.BlockSpec((1, 1, R), lambda i: (i, 0, 0)),
                  full, vec],
        out_specs=pl.BlockSpec((G, D), lambda i: (0, 0)),
        out_shape=jax.ShapeDtypeStruct((G, D), jnp.float32),
        scratch_shapes=[pltpu.VMEM((G, D), jnp.float32),
                        pltpu.VMEM((G, D), jnp.float32)],
    )(h, batch3, wc, bc)


def kernel(x, edge_index, batch,
           W1_0, b1_0, W2_0, b2_0, g_0, be_0,
           W1_1, b1_1, W2_1, b2_1, g_1, be_1,
           W1_2, b1_2, W2_2, b2_2, g_2, be_2,
           Wc, bc):
    params = [
        (W1_0, b1_0, W2_0, b2_0, g_0, be_0),
        (W1_1, b1_1, W2_1, b2_1, g_1, be_1),
        (W1_2, b1_2, W2_2, b2_2, g_2, be_2),
    ]
    pad = jnp.zeros((EP - E,), jnp.int32)
    src1 = jnp.concatenate([edge_index[0], pad + N])
    dst1 = jnp.concatenate([edge_index[1], pad])
    hrow_pad = jnp.zeros((NH - N, D), jnp.float32)
    zrows = jnp.zeros((RPT, D), jnp.float32)
    h = x
    for (w1, b1, w2, b2, g, be) in params:
        h_ext = jnp.concatenate([h, hrow_pad], axis=0)
        a0, a1 = _sc_scatter(h_ext, src1, dst1, zrows)
        t, s8, q8 = _mlp_stats(h, a0, a1, w1, b1, w2, b2)
        h = _bn_relu(t, s8, q8, g, be)
    batch3 = batch.reshape(GRID, 1, R)
    return _pool(h, batch3, Wc, bc)


# speedup vs baseline: 4.4558x; 4.0068x over previous
import functools

import jax
import jax.numpy as jnp
from jax import lax
from jax.experimental import pallas as pl
from jax.experimental.pallas import tpu as pltpu
from jax.experimental.pallas import tpu_sc as plsc

N = 10000
E = 320000
D = 128
G = 128

NW = 32
CH = 128
CPW = 80
EP = NW * CPW * CH
NTR = N + 128
RPT = 624
TAIL = N - 16 * RPT
HB = CPW // 2


def _sc_scatter(h, src1, dst1, zrows):
    mesh = plsc.VectorSubcoreMesh(core_axis_name="c", subcore_axis_name="s")

    @functools.partial(
        pl.kernel,
        mesh=mesh,
        out_type=(
            jax.ShapeDtypeStruct((N, D), jnp.float32),
            jax.ShapeDtypeStruct((N, D), jnp.float32),
        ),
        scratch_types=(
            [pltpu.VMEM((CH,), jnp.int32)] * 8
            + [pltpu.VMEM((CH, D), jnp.float32)] * 2
            + [pltpu.VMEM_SHARED((NTR, D), jnp.float32)]
            + [pltpu.SemaphoreType.DMA] * 6
        ),
    )
    def k(h_hbm, src_hbm, dst_hbm, z_hbm, out0, out1,
          si0, si1, si2, si3, di0, di1, di2, di3, buf0, buf1, agg_sh,
          is0, is1, is2, is3, gs0, gs1):
        c = lax.axis_index("c")
        s = lax.axis_index("s")
        wid = s * 2 + c
        sis = [si0, si1, si2, si3]
        dis = [di0, di1, di2, di3]
        bufs = [buf0, buf1]
        iss = [is0, is1, is2, is3]
        gss = [gs0, gs1]

        stripe = s * RPT
        pltpu.sync_copy(z_hbm, agg_sh.at[pl.ds(stripe, RPT)])

        @pl.when(s == 15)
        def _():
            pltpu.sync_copy(z_hbm.at[pl.ds(0, TAIL)],
                            agg_sh.at[pl.ds(16 * RPT, TAIL)])

        plsc.subcore_barrier()

        ebase = wid * (CPW * CH)

        def idx_start(m, t):
            pltpu.async_copy(src_hbm.at[pl.ds(ebase + m * CH, CH)],
                             sis[t], iss[t])
            pltpu.async_copy(dst_hbm.at[pl.ds(ebase + m * CH, CH)],
                             dis[t], iss[t])

        def idx_wait(m, t):
            pltpu.make_async_copy(src_hbm.at[pl.ds(ebase + m * CH, CH)],
                                  sis[t], iss[t]).wait()
            pltpu.make_async_copy(dst_hbm.at[pl.ds(ebase + m * CH, CH)],
                                  dis[t], iss[t]).wait()

        def gat_start(t, b):
            pltpu.async_copy(h_hbm.at[sis[t]], bufs[b], gss[b])

        def gat_wait(t, b):
            pltpu.make_async_copy(h_hbm.at[sis[t]], bufs[b], gss[b]).wait()

        for t in range(4):
            idx_start(t, t)
        idx_wait(0, 0)
        gat_start(0, 0)

        def body(j, carry):
            for kk in range(4):
                m = 4 * j + kk
                t = kk
                b = kk % 2
                tn = (kk + 1) % 4
                bn = (kk + 1) % 2

                @pl.when(m < CPW - 1)
                def _():
                    idx_wait(m + 1, tn)
                    gat_start(tn, bn)

                gat_wait(t, b)
                pltpu.sync_copy(bufs[b], agg_sh.at[dis[t]], add=True)

                @pl.when(m < CPW - 4)
                def _():
                    idx_start(m + 4, t)
            return carry

        lax.fori_loop(0, CPW // 4, body, 0)
        plsc.subcore_barrier()

        @pl.when(c == 0)
        def _():
            pltpu.sync_copy(agg_sh.at[pl.ds(stripe, RPT)],
                            out0.at[pl.ds(stripe, RPT)])

            @pl.when(s == 15)
            def _():
                pltpu.sync_copy(agg_sh.at[pl.ds(16 * RPT, TAIL)],
                                out0.at[pl.ds(16 * RPT, TAIL)])

        @pl.when(c == 1)
        def _():
            pltpu.sync_copy(agg_sh.at[pl.ds(stripe, RPT)],
                            out1.at[pl.ds(stripe, RPT)])

            @pl.when(s == 15)
            def _():
                pltpu.sync_copy(agg_sh.at[pl.ds(16 * RPT, TAIL)],
                                out1.at[pl.ds(16 * RPT, TAIL)])

    return k(h, src1, dst1, zrows)


R = 1000
GRID = N // R


def _mlp_stats_body(x_r, a0_r, a1_r, w1_r, b1_r, w2_r, b2_r,
                    t_r, sum_r, sq_r, accs, accq):
    i = pl.program_id(0)
    m = x_r[...] + a0_r[...] + a1_r[...]
    hmid = jnp.maximum(
        jnp.dot(m, w1_r[...], preferred_element_type=jnp.float32) + b1_r[...],
        0.0)
    t = jnp.dot(hmid, w2_r[...], preferred_element_type=jnp.float32) + b2_r[...]
    t_r[...] = t
    t3 = t.reshape(R // 8, 8, D)
    ps = jnp.sum(t3, axis=0)
    pq = jnp.sum(t3 * t3, axis=0)

    @pl.when(i == 0)
    def _():
        accs[...] = ps
        accq[...] = pq

    @pl.when(i > 0)
    def _():
        accs[...] += ps
        accq[...] += pq

    @pl.when(i == GRID - 1)
    def _():
        sum_r[...] = accs[...]
        sq_r[...] = accq[...]


def _mlp_stats(x, a0, a1, w1, b1, w2, b2):
    blk = pl.BlockSpec((R, D), lambda i: (i, 0))
    full = pl.BlockSpec((D, D), lambda i: (0, 0))
    vec = pl.BlockSpec((D,), lambda i: (0,))
    return pl.pallas_call(
        _mlp_stats_body,
        grid=(GRID,),
        in_specs=[blk, blk, blk, full, vec, full, vec],
        out_specs=[blk,
                   pl.BlockSpec((8, D), lambda i: (0, 0)),
                   pl.BlockSpec((8, D), lambda i: (0, 0))],
        out_shape=[jax.ShapeDtypeStruct((N, D), jnp.float32),
                   jax.ShapeDtypeStruct((8, D), jnp.float32),
                   jax.ShapeDtypeStruct((8, D), jnp.float32)],
        scratch_shapes=[pltpu.VMEM((8, D), jnp.float32),
                        pltpu.VMEM((8, D), jnp.float32)],
    )(x, a0, a1, w1, b1, w2, b2)


def _bn_relu_body(t_r, sum_r, sq_r, g_r, be_r, h_r):
    mean = jnp.sum(sum_r[...], axis=0) / N
    ex2 = jnp.sum(sq_r[...], axis=0) / N
    var = ex2 - mean * mean
    scale = lax.rsqrt(var + 1e-5) * g_r[...]
    h_r[...] = jnp.maximum((t_r[...] - mean) * scale + be_r[...], 0.0)


def _bn_relu(t, s8, q8, g, be):
    blk = pl.BlockSpec((R, D), lambda i: (i, 0))
    stat = pl.BlockSpec((8, D), lambda i: (0, 0))
    vec = pl.BlockSpec((D,), lambda i: (0,))
    return pl.pallas_call(
        _bn_relu_body,
        grid=(GRID,),
        in_specs=[blk, stat, stat, vec, vec],
        out_specs=blk,
        out_shape=jax.ShapeDtypeStruct((N, D), jnp.float32),
    )(t, s8, q8, g, be)


def _pool_body(h_r, b_r, wc_r, bc_r, out_r, accp, accc):
    i = pl.program_id(0)
    bvec = b_r[0, 0, :]
    gids = lax.broadcasted_iota(jnp.int32, (G, R), 0)
    mask = (gids == bvec[None, :]).astype(jnp.float32)
    ps = jnp.dot(mask, h_r[...], preferred_element_type=jnp.float32)
    pc = jnp.dot(mask, jnp.ones((R, D), jnp.float32),
                 preferred_element_type=jnp.float32)

    @pl.when(i == 0)
    def _():
        accp[...] = ps
        accc[...] = pc

    @pl.when(i > 0)
    def _():
        accp[...] += ps
        accc[...] += pc

    @pl.when(i == GRID - 1)
    def _():
        pooled = accp[...] / jnp.maximum(accc[...], 1.0)
        out_r[...] = (jnp.dot(pooled, wc_r[...],
                              preferred_element_type=jnp.float32) + bc_r[...])


def _pool(h, batch3, wc, bc):
    blk = pl.BlockSpec((R, D), lambda i: (i, 0))
    full = pl.BlockSpec((D, D), lambda i: (0, 0))
    vec = pl.BlockSpec((D,), lambda i: (0,))
    return pl.pallas_call(
        _pool_body,
        grid=(GRID,),
        in_specs=[blk,
                  pl.BlockSpec((1, 1, R), lambda i: (i, 0, 0)),
                  full, vec],
        out_specs=pl.BlockSpec((G, D), lambda i: (0, 0)),
        out_shape=jax.ShapeDtypeStruct((G, D), jnp.float32),
        scratch_shapes=[pltpu.VMEM((G, D), jnp.float32),
                        pltpu.VMEM((G, D), jnp.float32)],
    )(h, batch3, wc, bc)


def kernel(x, edge_index, batch,
           W1_0, b1_0, W2_0, b2_0, g_0, be_0,
           W1_1, b1_1, W2_1, b2_1, g_1, be_1,
           W1_2, b1_2, W2_2, b2_2, g_2, be_2,
           Wc, bc):
    params = [
        (W1_0, b1_0, W2_0, b2_0, g_0, be_0),
        (W1_1, b1_1, W2_1, b2_1, g_1, be_1),
        (W1_2, b1_2, W2_2, b2_2, g_2, be_2),
    ]
    pad = jnp.arange(EP - E, dtype=jnp.int32)
    src1 = jnp.concatenate([edge_index[0], pad % N])
    dst1 = jnp.concatenate([edge_index[1], N + pad % 128])
    zrows = jnp.zeros((RPT, D), jnp.float32)
    h = x
    for (w1, b1, w2, b2, g, be) in params:
        a0, a1 = _sc_scatter(h, src1, dst1, zrows)
        t, s8, q8 = _mlp_stats(h, a0, a1, w1, b1, w2, b2)
        h = _bn_relu(t, s8, q8, g, be)
    batch3 = batch.reshape(GRID, 1, R)
    return _pool(h, batch3, Wc, bc)
